# baseline (speedup 1.0000x reference)
import functools

import jax
import jax.numpy as jnp
from jax import lax
from jax.experimental import pallas as pl
from jax.experimental.pallas import tpu as pltpu
from jax.experimental.pallas import tpu_sc as plsc

NC = 2
NS = 16
NW = NC * NS
L = 16
CHUNK = 32
NHALF = CHUNK // L
GROWS = 4 * CHUNK


def _make_kernel(H: int, C: int, n_pad: int):
    cpw = n_pad // NW
    nchunks = cpw // CHUNK
    scale = jnp.float32(H / 128.0)
    hm1 = H - 1
    half_rows = H * H * H // 2
    ci = C // 2

    mesh = plsc.VectorSubcoreMesh(core_axis_name="c", subcore_axis_name="s")

    @functools.partial(
        pl.kernel,
        out_type=jax.ShapeDtypeStruct((n_pad, C), jnp.float32),
        mesh=mesh,
        compiler_params=pltpu.CompilerParams(needs_layout_passes=False),
        scratch_types=dict(
            xv=pltpu.VMEM((cpw,), jnp.float32),
            yv=pltpu.VMEM((cpw,), jnp.float32),
            zv=pltpu.VMEM((cpw,), jnp.float32),
            idx=[pltpu.VMEM((GROWS,), jnp.int32) for _ in range(2)],
            rows=[pltpu.VMEM((GROWS, 2 * ci), jnp.int32) for _ in range(2)],
            ob=[pltpu.VMEM((CHUNK, C), jnp.float32) for _ in range(2)],
            gsem=[pltpu.SemaphoreType.DMA for _ in range(2)],
            osem=[pltpu.SemaphoreType.DMA for _ in range(2)],
        ),
    )
    def k(img_hbm, x_hbm, y_hbm, z_hbm, out_hbm, *, xv, yv, zv, idx, rows,
          ob, gsem, osem):
        wid = lax.axis_index("s") * NC + lax.axis_index("c")
        base = wid * cpw
        iota = lax.iota(jnp.int32, L)

        pltpu.sync_copy(x_hbm.at[pl.ds(base, cpw)], xv)
        pltpu.sync_copy(y_hbm.at[pl.ds(base, cpw)], yv)
        pltpu.sync_copy(z_hbm.at[pl.ds(base, cpw)], zv)

        def axis_indices(v):
            i1 = v.astype(jnp.int32)
            f1 = i1.astype(jnp.float32)
            i2 = jnp.minimum(jnp.where(v > f1, i1 + 1, i1), hm1)
            return i1, i2

        def load_group_coords(off):
            xs = xv[pl.ds(off, L)] * scale
            ys = yv[pl.ds(off, L)] * scale
            zs = zv[pl.ds(off, L)] * scale
            return xs, ys, zs

        def compute_indices(c, idx_ref):
            for h in range(NHALF):
                xs, ys, zs = load_group_coords(c * CHUNK + h * L)
                xi1, xi2 = axis_indices(xs)
                yi1, yi2 = axis_indices(ys)
                zi1, _ = axis_indices(zs)
                zq = (lax.shift_right_logical(zi1, 1)
                      + (zi1 & 1) * half_rows)
                r11 = (xi1 * H + yi1) * (H // 2) + zq
                r21 = (xi2 * H + yi1) * (H // 2) + zq
                r12 = (xi1 * H + yi2) * (H // 2) + zq
                r22 = (xi2 * H + yi2) * (H // 2) + zq
                for g, r in enumerate((r11, r21, r12, r22)):
                    idx_ref[pl.ds(g * CHUNK + h * L, L)] = r

        def fire_gather(b):
            pltpu.async_copy(img_hbm.at[idx[b]], rows[b], gsem[b])

        def wait_gather(b):
            pltpu.make_async_copy(img_hbm.at[idx[b]], rows[b], gsem[b]).wait()

        def compute_chunk(c, rows_ref, ob_ref):
            splat_dn = lax.GatherDimensionNumbers(
                offset_dims=(), collapsed_slice_dims=(0,), start_index_map=(0,))

            def splat(v, p):
                return lax.gather(
                    v, (iota * 0 + p)[:, None], splat_dn, slice_sizes=(1,),
                    mode=lax.GatherScatterMode.PROMISE_IN_BOUNDS)

            for h in range(NHALF):
                xs, ys, zs = load_group_coords(c * CHUNK + h * L)
                xi1, xi2 = axis_indices(xs)
                yi1, yi2 = axis_indices(ys)
                zi1, zi2 = axis_indices(zs)
                wx = xs - xi1.astype(jnp.float32)
                wx2 = xi2.astype(jnp.float32) - xs
                wy = ys - yi1.astype(jnp.float32)
                wy2 = yi2.astype(jnp.float32) - ys
                z1f = zi1.astype(jnp.float32)
                z2f = zi2.astype(jnp.float32)
                zd = z2f - z1f
                wz = (zs - z1f) * zd
                wz2 = (z2f - zs) * zd
                w11 = wx2 * wy2
                w21 = wx * wy2
                w12 = wx2 * wy
                w22 = wx * wy
                ws = (w11 * wz2, w21 * wz2, w12 * wz2, w22 * wz2,
                      w11 * wz, w21 * wz, w12 * wz, w22 * wz)

                @plsc.parallel_loop(0, L, unroll=2)
                def _(p):
                    wp = [splat(w, p) for w in ws]
                    for cg in range(C // (2 * L)):
                        q = []
                        for zslot in range(2):
                            for g in range(4):
                                w16 = rows_ref[g * CHUNK + h * L + p,
                                               pl.ds(zslot * ci + cg * L, L)]
                                q.append(plsc.unpack(
                                    plsc.bitcast(w16, jnp.bfloat16),
                                    format=plsc.PackFormat.INTERLEAVED))
                        for half in range(2):
                            acc = (((q[0][half] * wp[0] + q[1][half] * wp[1])
                                    + (q[2][half] * wp[2] + q[3][half] * wp[3]))
                                   + ((q[4][half] * wp[4] + q[5][half] * wp[5])
                                      + (q[6][half] * wp[6] + q[7][half] * wp[7])))
                            ob_ref[h * L + p,
                                   pl.ds(cg * 2 * L + half * L, L)] = acc

        for b in range(2):
            compute_indices(b, idx[b])
            fire_gather(b)

        @pl.loop(0, nchunks, step=2)
        def _(c0):
            for b in range(2):
                c = c0 + b
                wait_gather(b)

                @pl.when(c >= 2)
                def _():
                    pltpu.make_async_copy(
                        ob[b], out_hbm.at[pl.ds(base + (c - 2) * CHUNK, CHUNK)],
                        osem[b]).wait()

                compute_chunk(c, rows[b], ob[b])
                pltpu.async_copy(
                    ob[b], out_hbm.at[pl.ds(base + c * CHUNK, CHUNK)], osem[b])

                @pl.when(c + 2 < nchunks)
                def _():
                    compute_indices(c + 2, idx[b])
                    fire_gather(b)

        for b in range(2):
            c = nchunks - 2 + b
            pltpu.make_async_copy(
                ob[b], out_hbm.at[pl.ds(base + c * CHUNK, CHUNK)],
                osem[b]).wait()

    return k


def kernel(image_features, graph_features):
    H = image_features.shape[1]
    C = image_features.shape[-1]
    nvox = H * H * H
    img = image_features.reshape(nvox, C)
    g = graph_features[0]
    n = g.shape[0]
    quantum = NW * CHUNK * 2
    n_pad = ((n + quantum - 1) // quantum) * quantum
    x = jnp.pad(g[:, 0], (0, n_pad - n), mode="wrap")
    y = jnp.pad(g[:, 1], (0, n_pad - n), mode="wrap")
    z = jnp.pad(g[:, 2], (0, n_pad - n), mode="wrap")
    blk = jnp.arange(C) // 32 * 32
    j = jnp.arange(C) % 32
    perm = blk + (j // 2) + 16 * (j % 2)
    img_bf = img.astype(jnp.bfloat16)[:, perm]
    f32pairs = jax.lax.bitcast_convert_type(
        img_bf.reshape(nvox, C // 2, 2), jnp.int32)
    even = f32pairs.reshape(nvox // 2, C)
    odd = jnp.concatenate(
        [f32pairs[1:], jnp.zeros((1, C // 2), jnp.int32)],
        axis=0).reshape(nvox // 2, C)
    table = jnp.concatenate([even, odd], axis=0)
    out = _make_kernel(H, C, n_pad)(table, x, y, z)
    return out[:n].reshape(1, n, C)


# speedup vs baseline: 2.5497x; 2.5497x over previous
import functools

import jax
import jax.numpy as jnp
from jax import lax
from jax.experimental import pallas as pl
from jax.experimental.pallas import tpu as pltpu
from jax.experimental.pallas import tpu_sc as plsc

NC = 2
NS = 16
NW = NC * NS
L = 16
CHUNK = 32
NHALF = CHUNK // L
GROWS = 4 * CHUNK


def _make_kernel(H: int, C: int, n_pad: int):
    cpw = n_pad // NW
    nchunks = cpw // CHUNK
    scale = jnp.float32(H / 128.0)
    hm1 = H - 1
    half_rows = H * H * H // 2
    ci = C // 2

    mesh = plsc.VectorSubcoreMesh(core_axis_name="c", subcore_axis_name="s")

    @functools.partial(
        pl.kernel,
        out_type=jax.ShapeDtypeStruct((n_pad, C), jnp.float32),
        mesh=mesh,
        compiler_params=pltpu.CompilerParams(needs_layout_passes=False),
        scratch_types=dict(
            xv=pltpu.VMEM((cpw,), jnp.float32),
            yv=pltpu.VMEM((cpw,), jnp.float32),
            zv=pltpu.VMEM((cpw,), jnp.float32),
            idx=[pltpu.VMEM((GROWS,), jnp.int32) for _ in range(2)],
            rows=[pltpu.VMEM((GROWS, 2 * ci), jnp.int32) for _ in range(2)],
            ob=[pltpu.VMEM((CHUNK, C), jnp.float32) for _ in range(2)],
            gsem=[pltpu.SemaphoreType.DMA for _ in range(2)],
            osem=[pltpu.SemaphoreType.DMA for _ in range(2)],
        ),
    )
    def k(img_hbm, x_hbm, y_hbm, z_hbm, out_hbm, *, xv, yv, zv, idx, rows,
          ob, gsem, osem):
        wid = lax.axis_index("s") * NC + lax.axis_index("c")
        base = wid * cpw
        iota = lax.iota(jnp.int32, L)

        pltpu.sync_copy(x_hbm.at[pl.ds(base, cpw)], xv)
        pltpu.sync_copy(y_hbm.at[pl.ds(base, cpw)], yv)
        pltpu.sync_copy(z_hbm.at[pl.ds(base, cpw)], zv)

        def axis_indices(v):
            i1 = v.astype(jnp.int32)
            f1 = i1.astype(jnp.float32)
            i2 = jnp.minimum(jnp.where(v > f1, i1 + 1, i1), hm1)
            return i1, i2

        def load_group_coords(off):
            xs = xv[pl.ds(off, L)] * scale
            ys = yv[pl.ds(off, L)] * scale
            zs = zv[pl.ds(off, L)] * scale
            return xs, ys, zs

        def compute_indices(c, idx_ref):
            for h in range(NHALF):
                xs, ys, zs = load_group_coords(c * CHUNK + h * L)
                xi1, xi2 = axis_indices(xs)
                yi1, yi2 = axis_indices(ys)
                zi1, _ = axis_indices(zs)
                r11 = (xi1 * H + yi1) * H + zi1
                r21 = (xi2 * H + yi1) * H + zi1
                r12 = (xi1 * H + yi2) * H + zi1
                r22 = (xi2 * H + yi2) * H + zi1
                for g, r in enumerate((r11, r21, r12, r22)):
                    idx_ref[pl.ds(g * CHUNK + h * L, L)] = r

        def fire_gather(b):
            pltpu.async_copy(img_hbm.at[idx[b]], rows[b], gsem[b])

        def wait_gather(b):
            pltpu.make_async_copy(img_hbm.at[idx[b]], rows[b], gsem[b]).wait()

        def compute_chunk(c, rows_ref, ob_ref):
            splat_dn = lax.GatherDimensionNumbers(
                offset_dims=(), collapsed_slice_dims=(0,), start_index_map=(0,))

            def splat(v, p):
                return lax.gather(
                    v, (iota * 0 + p)[:, None], splat_dn, slice_sizes=(1,),
                    mode=lax.GatherScatterMode.PROMISE_IN_BOUNDS)

            for h in range(NHALF):
                xs, ys, zs = load_group_coords(c * CHUNK + h * L)
                xi1, xi2 = axis_indices(xs)
                yi1, yi2 = axis_indices(ys)
                zi1, zi2 = axis_indices(zs)
                wx = xs - xi1.astype(jnp.float32)
                wx2 = xi2.astype(jnp.float32) - xs
                wy = ys - yi1.astype(jnp.float32)
                wy2 = yi2.astype(jnp.float32) - ys
                z1f = zi1.astype(jnp.float32)
                z2f = zi2.astype(jnp.float32)
                zd = z2f - z1f
                wz = (zs - z1f) * zd
                wz2 = (z2f - zs) * zd
                w11 = wx2 * wy2
                w21 = wx * wy2
                w12 = wx2 * wy
                w22 = wx * wy
                ws = (w11 * wz2, w21 * wz2, w12 * wz2, w22 * wz2,
                      w11 * wz, w21 * wz, w12 * wz, w22 * wz)

                @plsc.parallel_loop(0, L, unroll=2)
                def _(p):
                    wp = [splat(w, p) for w in ws]
                    hi_mask = jnp.int32(-65536)
                    for cg in range(C // (2 * L)):
                        q = []
                        for zslot in range(2):
                            for g in range(4):
                                w16 = rows_ref[g * CHUNK + h * L + p,
                                               pl.ds(zslot * ci + cg * L, L)]
                                q.append((
                                    plsc.bitcast(lax.shift_left(w16, 16),
                                                 jnp.float32),
                                    plsc.bitcast(w16 & hi_mask, jnp.float32)))
                        for half in range(2):
                            acc = (((q[0][half] * wp[0] + q[1][half] * wp[1])
                                    + (q[2][half] * wp[2] + q[3][half] * wp[3]))
                                   + ((q[4][half] * wp[4] + q[5][half] * wp[5])
                                      + (q[6][half] * wp[6] + q[7][half] * wp[7])))
                            ob_ref[h * L + p,
                                   pl.ds(half * (C // 2) + cg * L, L)] = acc

        for b in range(2):
            compute_indices(b, idx[b])
            fire_gather(b)

        @pl.loop(0, nchunks, step=2)
        def _(c0):
            for b in range(2):
                c = c0 + b
                wait_gather(b)

                @pl.when(c >= 2)
                def _():
                    pltpu.make_async_copy(
                        ob[b], out_hbm.at[pl.ds(base + (c - 2) * CHUNK, CHUNK)],
                        osem[b]).wait()

                compute_chunk(c, rows[b], ob[b])
                pltpu.async_copy(
                    ob[b], out_hbm.at[pl.ds(base + c * CHUNK, CHUNK)], osem[b])

                @pl.when(c + 2 < nchunks)
                def _():
                    compute_indices(c + 2, idx[b])
                    fire_gather(b)

        for b in range(2):
            c = nchunks - 2 + b
            pltpu.make_async_copy(
                ob[b], out_hbm.at[pl.ds(base + c * CHUNK, CHUNK)],
                osem[b]).wait()

    return k


def kernel(image_features, graph_features):
    H = image_features.shape[1]
    C = image_features.shape[-1]
    nvox = H * H * H
    img = image_features.reshape(nvox, C)
    g = graph_features[0]
    n = g.shape[0]
    quantum = NW * CHUNK * 2
    n_pad = ((n + quantum - 1) // quantum) * quantum
    x = jnp.pad(g[:, 0], (0, n_pad - n), mode="wrap")
    y = jnp.pad(g[:, 1], (0, n_pad - n), mode="wrap")
    z = jnp.pad(g[:, 2], (0, n_pad - n), mode="wrap")
    b16 = lax.bitcast_convert_type(img.astype(jnp.bfloat16), jnp.uint16)
    w_all = (b16[:, :C // 2].astype(jnp.uint32)
             | (b16[:, C // 2:].astype(jnp.uint32) << 16))
    w_next = jnp.concatenate(
        [w_all[1:], jnp.zeros((1, C // 2), jnp.uint32)], axis=0)
    table = lax.bitcast_convert_type(
        jnp.concatenate([w_all, w_next], axis=1), jnp.int32)
    out = _make_kernel(H, C, n_pad)(table, x, y, z)
    return out[:n].reshape(1, n, C)


# speedup vs baseline: 2.6233x; 1.0289x over previous
import functools

import jax
import jax.numpy as jnp
from jax import lax
from jax.experimental import pallas as pl
from jax.experimental.pallas import tpu as pltpu
from jax.experimental.pallas import tpu_sc as plsc

NC = 2
NS = 16
NW = NC * NS
L = 16
CHUNK = 32
NHALF = CHUNK // L
GROWS = 4 * CHUNK


def _make_kernel(H: int, C: int, n_pad: int):
    cpw = n_pad // NW
    nchunks = cpw // CHUNK
    scale = jnp.float32(H / 128.0)
    hm1 = H - 1
    half_rows = H * H * H // 2
    ci = C // 2

    mesh = plsc.VectorSubcoreMesh(core_axis_name="c", subcore_axis_name="s")

    @functools.partial(
        pl.kernel,
        out_type=jax.ShapeDtypeStruct((n_pad, C), jnp.float32),
        mesh=mesh,
        compiler_params=pltpu.CompilerParams(needs_layout_passes=False),
        scratch_types=dict(
            xv=pltpu.VMEM((cpw,), jnp.float32),
            yv=pltpu.VMEM((cpw,), jnp.float32),
            zv=pltpu.VMEM((cpw,), jnp.float32),
            idx=[pltpu.VMEM((GROWS,), jnp.int32) for _ in range(2)],
            rows=[pltpu.VMEM((GROWS, 2 * ci), jnp.int32) for _ in range(2)],
            ob=[pltpu.VMEM((CHUNK, C), jnp.float32) for _ in range(2)],
            gsem=[pltpu.SemaphoreType.DMA for _ in range(2)],
            osem=[pltpu.SemaphoreType.DMA for _ in range(2)],
        ),
    )
    def k(img_hbm, x_hbm, y_hbm, z_hbm, out_hbm, *, xv, yv, zv, idx, rows,
          ob, gsem, osem):
        wid = lax.axis_index("s") * NC + lax.axis_index("c")
        base = wid * cpw
        iota = lax.iota(jnp.int32, L)

        pltpu.sync_copy(x_hbm.at[pl.ds(base, cpw)], xv)
        pltpu.sync_copy(y_hbm.at[pl.ds(base, cpw)], yv)
        pltpu.sync_copy(z_hbm.at[pl.ds(base, cpw)], zv)

        def axis_indices(v):
            i1 = v.astype(jnp.int32)
            f1 = i1.astype(jnp.float32)
            i2 = jnp.minimum(jnp.where(v > f1, i1 + 1, i1), hm1)
            return i1, i2

        def load_group_coords(off):
            xs = xv[pl.ds(off, L)] * scale
            ys = yv[pl.ds(off, L)] * scale
            zs = zv[pl.ds(off, L)] * scale
            return xs, ys, zs

        def compute_indices(c, idx_ref):
            for h in range(NHALF):
                xs, ys, zs = load_group_coords(c * CHUNK + h * L)
                xi1, xi2 = axis_indices(xs)
                yi1, yi2 = axis_indices(ys)
                zi1, _ = axis_indices(zs)
                r11 = (xi1 * H + yi1) * H + zi1
                r21 = (xi2 * H + yi1) * H + zi1
                r12 = (xi1 * H + yi2) * H + zi1
                r22 = (xi2 * H + yi2) * H + zi1
                for g, r in enumerate((r11, r21, r12, r22)):
                    idx_ref[pl.ds(g * CHUNK + h * L, L)] = r

        def fire_gather(b):
            pltpu.async_copy(img_hbm.at[idx[b]], rows[b], gsem[b])

        def wait_gather(b):
            pltpu.make_async_copy(img_hbm.at[idx[b]], rows[b], gsem[b]).wait()

        def compute_chunk(c, rows_ref, ob_ref):
            splat_dn = lax.GatherDimensionNumbers(
                offset_dims=(), collapsed_slice_dims=(0,), start_index_map=(0,))

            def splat(v, p):
                return lax.gather(
                    v, (iota * 0 + p)[:, None], splat_dn, slice_sizes=(1,),
                    mode=lax.GatherScatterMode.PROMISE_IN_BOUNDS)

            for h in range(NHALF):
                xs, ys, zs = load_group_coords(c * CHUNK + h * L)
                xi1, xi2 = axis_indices(xs)
                yi1, yi2 = axis_indices(ys)
                zi1, zi2 = axis_indices(zs)
                wx = xs - xi1.astype(jnp.float32)
                wx2 = xi2.astype(jnp.float32) - xs
                wy = ys - yi1.astype(jnp.float32)
                wy2 = yi2.astype(jnp.float32) - ys
                z1f = zi1.astype(jnp.float32)
                z2f = zi2.astype(jnp.float32)
                zd = z2f - z1f
                wz = (zs - z1f) * zd
                wz2 = (z2f - zs) * zd
                w11 = wx2 * wy2
                w21 = wx * wy2
                w12 = wx2 * wy
                w22 = wx * wy
                ws = (w11 * wz2, w21 * wz2, w12 * wz2, w22 * wz2,
                      w11 * wz, w21 * wz, w12 * wz, w22 * wz)

                @plsc.parallel_loop(0, L, unroll=2)
                def _(p):
                    wp = [splat(w, p) for w in ws]
                    hi_mask = jnp.int32(-65536)
                    for cg in range(C // L):
                        w16 = [rows_ref[g * CHUNK + h * L + p,
                                        pl.ds(cg * L, L)] for g in range(4)]
                        q1 = [plsc.bitcast(lax.shift_left(w, 16), jnp.float32)
                              for w in w16]
                        q2 = [plsc.bitcast(w & hi_mask, jnp.float32)
                              for w in w16]
                        acc = (((q1[0] * wp[0] + q1[1] * wp[1])
                                + (q1[2] * wp[2] + q1[3] * wp[3]))
                               + ((q2[0] * wp[4] + q2[1] * wp[5])
                                  + (q2[2] * wp[6] + q2[3] * wp[7])))
                        ob_ref[h * L + p, pl.ds(cg * L, L)] = acc

        for b in range(2):
            compute_indices(b, idx[b])
            fire_gather(b)

        @pl.loop(0, nchunks, step=2)
        def _(c0):
            for b in range(2):
                c = c0 + b
                wait_gather(b)

                @pl.when(c >= 2)
                def _():
                    pltpu.make_async_copy(
                        ob[b], out_hbm.at[pl.ds(base + (c - 2) * CHUNK, CHUNK)],
                        osem[b]).wait()

                compute_chunk(c, rows[b], ob[b])
                pltpu.async_copy(
                    ob[b], out_hbm.at[pl.ds(base + c * CHUNK, CHUNK)], osem[b])

                @pl.when(c + 2 < nchunks)
                def _():
                    compute_indices(c + 2, idx[b])
                    fire_gather(b)

        for b in range(2):
            c = nchunks - 2 + b
            pltpu.make_async_copy(
                ob[b], out_hbm.at[pl.ds(base + c * CHUNK, CHUNK)],
                osem[b]).wait()

    return k


def kernel(image_features, graph_features):
    H = image_features.shape[1]
    C = image_features.shape[-1]
    nvox = H * H * H
    img = image_features.reshape(nvox, C)
    g = graph_features[0]
    n = g.shape[0]
    quantum = NW * CHUNK * 2
    n_pad = ((n + quantum - 1) // quantum) * quantum
    x = jnp.pad(g[:, 0], (0, n_pad - n), mode="wrap")
    y = jnp.pad(g[:, 1], (0, n_pad - n), mode="wrap")
    z = jnp.pad(g[:, 2], (0, n_pad - n), mode="wrap")
    u = lax.bitcast_convert_type(img, jnp.uint32)
    s16 = jnp.uint32(16)
    t = u + jnp.uint32(0x7FFF) + (lax.shift_right_logical(u, s16)
                                  & jnp.uint32(1))
    t_next = jnp.concatenate(
        [t[1:], jnp.zeros((1, C), jnp.uint32)], axis=0)
    table = lax.bitcast_convert_type(
        lax.shift_right_logical(t, s16) | (t_next & jnp.uint32(0xFFFF0000)),
        jnp.int32)
    out = _make_kernel(H, C, n_pad)(table, x, y, z)
    return out[:n].reshape(1, n, C)


# speedup vs baseline: 5.5188x; 2.1038x over previous
import functools

import jax
import jax.numpy as jnp
from jax import lax
from jax.experimental import pallas as pl
from jax.experimental.pallas import tpu as pltpu
from jax.experimental.pallas import tpu_sc as plsc

NC = 2
NS = 16
NW = NC * NS
L = 16
CHUNK = 16
NHALF = CHUNK // L
GROWS = 8 * L


def _make_kernel(H: int, C: int, n_pad: int):
    cpw = n_pad // NW
    nchunks = cpw // CHUNK
    scale = jnp.float32(H / 128.0)
    hm1 = H - 1

    mesh = plsc.VectorSubcoreMesh(core_axis_name="c", subcore_axis_name="s")

    @functools.partial(
        pl.kernel,
        out_type=jax.ShapeDtypeStruct((n_pad, C), jnp.float32),
        mesh=mesh,
        compiler_params=pltpu.CompilerParams(needs_layout_passes=False),
        scratch_types=dict(
            xv=pltpu.VMEM((cpw,), jnp.float32),
            yv=pltpu.VMEM((cpw,), jnp.float32),
            zv=pltpu.VMEM((cpw,), jnp.float32),
            idx=[pltpu.VMEM((NHALF, GROWS), jnp.int32) for _ in range(2)],
            rows=[pltpu.VMEM((NHALF * GROWS, C), jnp.float32) for _ in range(2)],
            ob=[pltpu.VMEM((CHUNK, C), jnp.float32) for _ in range(2)],
            gsem=[pltpu.SemaphoreType.DMA for _ in range(2)],
            osem=[pltpu.SemaphoreType.DMA for _ in range(2)],
        ),
    )
    def k(img_hbm, x_hbm, y_hbm, z_hbm, out_hbm, *, xv, yv, zv, idx, rows,
          ob, gsem, osem):
        wid = lax.axis_index("s") * NC + lax.axis_index("c")
        base = wid * cpw
        iota = lax.iota(jnp.int32, L)

        pltpu.sync_copy(x_hbm.at[pl.ds(base, cpw)], xv)
        pltpu.sync_copy(y_hbm.at[pl.ds(base, cpw)], yv)
        pltpu.sync_copy(z_hbm.at[pl.ds(base, cpw)], zv)

        def axis_indices(v):
            i1 = v.astype(jnp.int32)
            f1 = i1.astype(jnp.float32)
            i2 = jnp.minimum(jnp.where(v > f1, i1 + 1, i1), hm1)
            return i1, i2

        def load_group_coords(off):
            xs = xv[pl.ds(off, L)] * scale
            ys = yv[pl.ds(off, L)] * scale
            zs = zv[pl.ds(off, L)] * scale
            return xs, ys, zs

        def compute_indices(c, idx_ref):
            for h in range(NHALF):
                xs, ys, zs = load_group_coords(c * CHUNK + h * L)
                xi1, xi2 = axis_indices(xs)
                yi1, yi2 = axis_indices(ys)
                zi1, zi2 = axis_indices(zs)
                r11 = (xi1 * H + yi1) * H
                r21 = (xi2 * H + yi1) * H
                r12 = (xi1 * H + yi2) * H
                r22 = (xi2 * H + yi2) * H
                for g, r in enumerate(
                        (r11 + zi1, r21 + zi1, r12 + zi1, r22 + zi1,
                         r11 + zi2, r21 + zi2, r12 + zi2, r22 + zi2)):
                    idx_ref[h, pl.ds(g * L, L)] = r

        def fire_gather(b):
            for h in range(NHALF):
                pltpu.async_copy(img_hbm.at[idx[b].at[h]],
                                 rows[b].at[pl.ds(h * GROWS, GROWS)], gsem[b])

        def wait_gather(b):
            for h in range(NHALF):
                pltpu.make_async_copy(img_hbm.at[idx[b].at[h]],
                                      rows[b].at[pl.ds(h * GROWS, GROWS)],
                                      gsem[b]).wait()

        def compute_chunk(c, rows_ref, ob_ref):
            splat_dn = lax.GatherDimensionNumbers(
                offset_dims=(), collapsed_slice_dims=(0,), start_index_map=(0,))

            def splat(v, p):
                return lax.gather(
                    v, (iota * 0 + p)[:, None], splat_dn, slice_sizes=(1,),
                    mode=lax.GatherScatterMode.PROMISE_IN_BOUNDS)

            for h in range(NHALF):
                xs, ys, zs = load_group_coords(c * CHUNK + h * L)
                xi1, xi2 = axis_indices(xs)
                yi1, yi2 = axis_indices(ys)
                zi1, zi2 = axis_indices(zs)
                wx = xs - xi1.astype(jnp.float32)
                wx2 = xi2.astype(jnp.float32) - xs
                wy = ys - yi1.astype(jnp.float32)
                wy2 = yi2.astype(jnp.float32) - ys
                wz = zs - zi1.astype(jnp.float32)
                wz2 = zi2.astype(jnp.float32) - zs
                w11 = wx2 * wy2
                w21 = wx * wy2
                w12 = wx2 * wy
                w22 = wx * wy
                ws = (w11 * wz2, w21 * wz2, w12 * wz2, w22 * wz2,
                      w11 * wz, w21 * wz, w12 * wz, w22 * wz)

                @plsc.parallel_loop(0, L, unroll=2)
                def _(p):
                    wp = [splat(w, p) for w in ws]
                    for cg in range(C // L):
                        s = pl.ds(cg * L, L)
                        q = [rows_ref[h * GROWS + g * L + p, s]
                             for g in range(8)]
                        acc = (((q[0] * wp[0] + q[1] * wp[1])
                                + (q[2] * wp[2] + q[3] * wp[3]))
                               + ((q[4] * wp[4] + q[5] * wp[5])
                                  + (q[6] * wp[6] + q[7] * wp[7])))
                        ob_ref[h * L + p, s] = acc

        for b in range(2):
            compute_indices(b, idx[b])
            fire_gather(b)

        @pl.loop(0, nchunks, step=2)
        def _(c0):
            for b in range(2):
                c = c0 + b
                wait_gather(b)

                @pl.when(c >= 2)
                def _():
                    pltpu.make_async_copy(
                        ob[b], out_hbm.at[pl.ds(base + (c - 2) * CHUNK, CHUNK)],
                        osem[b]).wait()

                compute_chunk(c, rows[b], ob[b])
                pltpu.async_copy(
                    ob[b], out_hbm.at[pl.ds(base + c * CHUNK, CHUNK)], osem[b])

                @pl.when(c + 2 < nchunks)
                def _():
                    compute_indices(c + 2, idx[b])
                    fire_gather(b)

        for b in range(2):
            c = nchunks - 2 + b
            pltpu.make_async_copy(
                ob[b], out_hbm.at[pl.ds(base + c * CHUNK, CHUNK)],
                osem[b]).wait()

    return k


def kernel(image_features, graph_features):
    H = image_features.shape[1]
    C = image_features.shape[-1]
    img = image_features.reshape(H * H * H, C)
    g = graph_features[0]
    n = g.shape[0]
    quantum = NW * CHUNK * 2
    n_pad = ((n + quantum - 1) // quantum) * quantum
    x = jnp.pad(g[:, 0], (0, n_pad - n), mode="wrap")
    y = jnp.pad(g[:, 1], (0, n_pad - n), mode="wrap")
    z = jnp.pad(g[:, 2], (0, n_pad - n), mode="wrap")
    out = _make_kernel(H, C, n_pad)(img, x, y, z)
    return out[:n].reshape(1, n, C)
